# baseline (speedup 1.0000x reference)
import functools

import jax
import jax.numpy as jnp
from jax import lax
from jax.experimental import pallas as pl
from jax.experimental.pallas import tpu as pltpu
from jax.experimental.pallas import tpu_sc as plsc

BATCH = 16384
EMB = 64
XCOL = 16
N_NUM = 8
N_ACT = 256
RATING_MIN = 1.0
RATING_MAX = 5.0

_info = plsc.get_sparse_core_info()
_NC, _NS, _L = _info.num_cores, _info.num_subcores, _info.num_lanes
_NW = _NC * _NS
_BPW = BATCH // _NW


def _sc_gather_body(x_hbm, ut_hbm, it_hbm, eu_hbm, ei_hbm,
                    x_v, uidx_v, iidx_v, urows_v, irows_v, sem_u, sem_i):
    wid = lax.axis_index("s") * _NC + lax.axis_index("c")
    base = wid * _BPW
    pltpu.sync_copy(x_hbm.at[pl.ds(base, _BPW), pl.ds(0, XCOL)], x_v)
    iota = lax.broadcasted_iota(jnp.int32, (_L,), 0)
    zero = jnp.zeros((_L,), jnp.int32)
    one = jnp.ones((_L,), jnp.int32)
    for j in range(_BPW // _L):
        rows = j * _L + iota
        uidx_v[pl.ds(j * _L, _L)] = plsc.bitcast(
            plsc.load_gather(x_v, [rows, zero]), jnp.int32)
        iidx_v[pl.ds(j * _L, _L)] = plsc.bitcast(
            plsc.load_gather(x_v, [rows, one]), jnp.int32)
    cu = pltpu.async_copy(ut_hbm.at[uidx_v], urows_v, sem_u)
    ci = pltpu.async_copy(it_hbm.at[iidx_v], irows_v, sem_i)
    cu.wait()
    ci.wait()
    pltpu.sync_copy(urows_v, eu_hbm.at[pl.ds(base, _BPW)])
    pltpu.sync_copy(irows_v, ei_hbm.at[pl.ds(base, _BPW)])


_sc_gather = functools.partial(
    pl.kernel,
    mesh=plsc.VectorSubcoreMesh(core_axis_name="c", subcore_axis_name="s"),
    compiler_params=pltpu.CompilerParams(use_tc_tiling_on_sc=False,
                                         needs_layout_passes=False),
    out_type=[
        jax.ShapeDtypeStruct((BATCH, EMB), jnp.bfloat16),
        jax.ShapeDtypeStruct((BATCH, EMB), jnp.bfloat16),
    ],
    scratch_types=[
        pltpu.VMEM((_BPW, XCOL), jnp.float32),
        pltpu.VMEM((_BPW,), jnp.int32),
        pltpu.VMEM((_BPW,), jnp.int32),
        pltpu.VMEM((_BPW, EMB), jnp.bfloat16),
        pltpu.VMEM((_BPW, EMB), jnp.bfloat16),
        pltpu.SemaphoreType.DMA,
        pltpu.SemaphoreType.DMA,
    ],
)(_sc_gather_body)


_BT = 2048


def _mlp_body(eu_ref, ei_ref, nf_ref, w1u_ref, w1i_ref, w1n_ref, b1_ref,
              w2_ref, b2_ref, out_ref):
    h = jnp.dot(eu_ref[...], w1u_ref[...], preferred_element_type=jnp.float32)
    h += jnp.dot(ei_ref[...], w1i_ref[...], preferred_element_type=jnp.float32)
    h += jnp.dot(nf_ref[...], w1n_ref[...], preferred_element_type=jnp.float32)
    h += b1_ref[...]
    h = jnp.maximum(h, 0.0)
    o = jnp.dot(h, w2_ref[...], preferred_element_type=jnp.float32)
    o += b2_ref[...]
    o = 1.0 / (1.0 + jnp.exp(-o))
    out_ref[...] = o * (RATING_MAX - RATING_MIN) + RATING_MIN


def _mlp(eu, ei, nf, w1u, w1i, w1n, b1, w2, b2):
    grid = (BATCH // _BT,)
    args = (eu, ei, nf, w1u, w1i, w1n, b1, w2, b2)
    bspec_b = lambda shape: pl.BlockSpec((_BT,) + shape[1:],
                                         lambda i: (i,) + (0,) * (len(shape) - 1))
    full = lambda shape: pl.BlockSpec(shape, lambda i: (0,) * len(shape))
    in_specs = [bspec_b(a.shape) for a in args[:3]]
    in_specs += [full(a.shape) for a in args[3:]]
    return pl.pallas_call(
        _mlp_body,
        grid=grid,
        in_specs=in_specs,
        out_specs=pl.BlockSpec((_BT, 1), lambda i: (i, 0)),
        out_shape=jax.ShapeDtypeStruct((BATCH, 1), jnp.float32),
    )(*args)


def kernel(X, user_table, item_table, W1, b1, W2, b2):
    nf = X[:, 2:].astype(jnp.float32)
    xpad = jnp.pad(lax.bitcast_convert_type(X, jnp.float32),
                   ((0, 0), (0, 128 - X.shape[1])))
    eu, ei = _sc_gather(xpad, user_table.astype(jnp.bfloat16),
                        item_table.astype(jnp.bfloat16))
    w1u = W1[:EMB].astype(jnp.bfloat16)
    w1i = W1[EMB:2 * EMB].astype(jnp.bfloat16)
    w1n = W1[2 * EMB:]
    return _mlp(eu, ei, nf, w1u, w1i, w1n, b1.reshape(1, N_ACT), W2,
                b2.reshape(1, 1))


# speedup vs baseline: 1.3025x; 1.3025x over previous
import functools

import jax
import jax.numpy as jnp
from jax import lax
from jax.experimental import pallas as pl
from jax.experimental.pallas import tpu as pltpu
from jax.experimental.pallas import tpu_sc as plsc

BATCH = 16384
EMB = 64
N_NUM = 8
N_ACT = 256
RATING_MIN = 1.0
RATING_MAX = 5.0

_info = plsc.get_sparse_core_info()
_NC, _NS = _info.num_cores, _info.num_subcores
_NW = _NC * _NS
_BPW = BATCH // _NW


def _sc_gather_body(ut_hbm, it_hbm, uidx_hbm, iidx_hbm, eu_hbm, ei_hbm,
                    uidx_v, urows_v, iidx_v, irows_v, sem_u, sem_i):
    wid = lax.axis_index("s") * _NC + lax.axis_index("c")
    base = wid * _BPW
    pltpu.sync_copy(uidx_hbm.at[pl.ds(base, _BPW)], uidx_v)
    pltpu.sync_copy(iidx_hbm.at[pl.ds(base, _BPW)], iidx_v)
    cu = pltpu.async_copy(ut_hbm.at[uidx_v], urows_v, sem_u)
    ci = pltpu.async_copy(it_hbm.at[iidx_v], irows_v, sem_i)
    cu.wait()
    ci.wait()
    pltpu.sync_copy(urows_v, eu_hbm.at[pl.ds(base, _BPW)])
    pltpu.sync_copy(irows_v, ei_hbm.at[pl.ds(base, _BPW)])


_sc_gather = functools.partial(
    pl.kernel,
    mesh=plsc.VectorSubcoreMesh(core_axis_name="c", subcore_axis_name="s"),
    compiler_params=pltpu.CompilerParams(use_tc_tiling_on_sc=False),
    out_type=[
        jax.ShapeDtypeStruct((BATCH, EMB), jnp.float32),
        jax.ShapeDtypeStruct((BATCH, EMB), jnp.float32),
    ],
    scratch_types=[
        pltpu.VMEM((_BPW,), jnp.int32),
        pltpu.VMEM((_BPW, EMB), jnp.float32),
        pltpu.VMEM((_BPW,), jnp.int32),
        pltpu.VMEM((_BPW, EMB), jnp.float32),
        pltpu.SemaphoreType.DMA,
        pltpu.SemaphoreType.DMA,
    ],
)(_sc_gather_body)


_BT = 2048


def _mlp_body(eu_ref, ei_ref, nf_ref, w1u_ref, w1i_ref, w1n_ref, b1_ref,
              w2_ref, b2_ref, out_ref):
    h = jnp.dot(eu_ref[...], w1u_ref[...], preferred_element_type=jnp.float32)
    h += jnp.dot(ei_ref[...], w1i_ref[...], preferred_element_type=jnp.float32)
    h += jnp.dot(nf_ref[...], w1n_ref[...], preferred_element_type=jnp.float32)
    h += b1_ref[...]
    h = jnp.maximum(h, 0.0)
    o = jnp.dot(h, w2_ref[...], preferred_element_type=jnp.float32)
    o += b2_ref[...]
    o = 1.0 / (1.0 + jnp.exp(-o))
    out_ref[...] = o * (RATING_MAX - RATING_MIN) + RATING_MIN


def _mlp(eu, ei, nf, w1u, w1i, w1n, b1, w2, b2):
    grid = (BATCH // _BT,)
    args = (eu, ei, nf, w1u, w1i, w1n, b1, w2, b2)
    bspec_b = lambda shape: pl.BlockSpec((_BT,) + shape[1:],
                                         lambda i: (i,) + (0,) * (len(shape) - 1))
    full = lambda shape: pl.BlockSpec(shape, lambda i: (0,) * len(shape))
    in_specs = [bspec_b(a.shape) for a in args[:3]]
    in_specs += [full(a.shape) for a in args[3:]]
    return pl.pallas_call(
        _mlp_body,
        grid=grid,
        in_specs=in_specs,
        out_specs=pl.BlockSpec((_BT, 1), lambda i: (i, 0)),
        out_shape=jax.ShapeDtypeStruct((BATCH, 1), jnp.float32),
    )(*args)


def kernel(X, user_table, item_table, W1, b1, W2, b2):
    uidx = X[:, 0]
    iidx = X[:, 1]
    nf = X[:, 2:].astype(jnp.float32)
    eu, ei = _sc_gather(user_table, item_table, uidx, iidx)
    w1u = W1[:EMB]
    w1i = W1[EMB:2 * EMB]
    w1n = W1[2 * EMB:]
    return _mlp(eu, ei, nf, w1u, w1i, w1n, b1.reshape(1, N_ACT), W2,
                b2.reshape(1, 1))


# speedup vs baseline: 1.3096x; 1.0055x over previous
import functools

import jax
import jax.numpy as jnp
from jax import lax
from jax.experimental import pallas as pl
from jax.experimental.pallas import tpu as pltpu
from jax.experimental.pallas import tpu_sc as plsc

BATCH = 16384
EMB = 64
N_NUM = 8
N_ACT = 256
RATING_MIN = 1.0
RATING_MAX = 5.0

_info = plsc.get_sparse_core_info()
_NC, _NS = _info.num_cores, _info.num_subcores
_NW = _NC * _NS
_BPW = BATCH // _NW


def _sc_gather_body(t_hbm, idx_hbm, out_hbm, idx_v, rows_v, sem):
    wid = lax.axis_index("s") * _NC + lax.axis_index("c")
    base = wid * _BPW
    pltpu.sync_copy(idx_hbm.at[pl.ds(base, _BPW)], idx_v)
    pltpu.async_copy(t_hbm.at[idx_v], rows_v, sem).wait()
    pltpu.sync_copy(rows_v, out_hbm.at[pl.ds(base, _BPW)])


_sc_gather = functools.partial(
    pl.kernel,
    mesh=plsc.VectorSubcoreMesh(core_axis_name="c", subcore_axis_name="s"),
    compiler_params=pltpu.CompilerParams(use_tc_tiling_on_sc=False),
    out_type=jax.ShapeDtypeStruct((BATCH, EMB), jnp.float32),
    scratch_types=[
        pltpu.VMEM((_BPW,), jnp.int32),
        pltpu.VMEM((_BPW, EMB), jnp.float32),
        pltpu.SemaphoreType.DMA,
    ],
)(_sc_gather_body)


_BT = 2048


def _mlp_body(eu_ref, ei_ref, nf_ref, w1u_ref, w1i_ref, w1n_ref, b1_ref,
              w2_ref, b2_ref, out_ref):
    h = jnp.dot(eu_ref[...], w1u_ref[...], preferred_element_type=jnp.float32)
    h += jnp.dot(ei_ref[...], w1i_ref[...], preferred_element_type=jnp.float32)
    h += jnp.dot(nf_ref[...], w1n_ref[...], preferred_element_type=jnp.float32)
    h += b1_ref[...]
    h = jnp.maximum(h, 0.0)
    o = jnp.dot(h, w2_ref[...], preferred_element_type=jnp.float32)
    o += b2_ref[...]
    o = 1.0 / (1.0 + jnp.exp(-o))
    out_ref[...] = o * (RATING_MAX - RATING_MIN) + RATING_MIN


def _mlp(eu, ei, nf, w1u, w1i, w1n, b1, w2, b2):
    grid = (BATCH // _BT,)
    args = (eu, ei, nf, w1u, w1i, w1n, b1, w2, b2)
    bspec_b = lambda shape: pl.BlockSpec((_BT,) + shape[1:],
                                         lambda i: (i,) + (0,) * (len(shape) - 1))
    full = lambda shape: pl.BlockSpec(shape, lambda i: (0,) * len(shape))
    in_specs = [bspec_b(a.shape) for a in args[:3]]
    in_specs += [full(a.shape) for a in args[3:]]
    return pl.pallas_call(
        _mlp_body,
        grid=grid,
        in_specs=in_specs,
        out_specs=pl.BlockSpec((_BT, 1), lambda i: (i, 0)),
        out_shape=jax.ShapeDtypeStruct((BATCH, 1), jnp.float32),
    )(*args)


def kernel(X, user_table, item_table, W1, b1, W2, b2):
    uidx = X[:, 0]
    iidx = X[:, 1]
    nf = X[:, 2:].astype(jnp.float32)
    eu = _sc_gather(user_table, uidx)
    ei = _sc_gather(item_table, iidx)
    w1u = W1[:EMB]
    w1i = W1[EMB:2 * EMB]
    w1n = W1[2 * EMB:]
    return _mlp(eu, ei, nf, w1u, w1i, w1n, b1.reshape(1, N_ACT), W2,
                b2.reshape(1, 1))


# speedup vs baseline: 1.3814x; 1.0548x over previous
import functools

import jax
import jax.numpy as jnp
from jax import lax
from jax.experimental import pallas as pl
from jax.experimental.pallas import tpu as pltpu
from jax.experimental.pallas import tpu_sc as plsc

BATCH = 16384
EMB = 64
N_NUM = 8
N_ACT = 256
RATING_MIN = 1.0
RATING_MAX = 5.0

_info = plsc.get_sparse_core_info()
_NC, _NS = _info.num_cores, _info.num_subcores
_NW = _NC * _NS
_BPW = BATCH // _NW


def _sc_gather_body(t_hbm, idx_hbm, out_hbm, idx_v, rows_v, sem):
    wid = lax.axis_index("s") * _NC + lax.axis_index("c")
    base = wid * _BPW
    pltpu.sync_copy(idx_hbm.at[pl.ds(base, _BPW)], idx_v)
    pltpu.async_copy(t_hbm.at[idx_v], rows_v, sem).wait()
    pltpu.sync_copy(rows_v, out_hbm.at[pl.ds(base, _BPW)])


_sc_gather = functools.partial(
    pl.kernel,
    mesh=plsc.VectorSubcoreMesh(core_axis_name="c", subcore_axis_name="s"),
    compiler_params=pltpu.CompilerParams(use_tc_tiling_on_sc=False),
    out_type=jax.ShapeDtypeStruct((BATCH, EMB), jnp.float32),
    scratch_types=[
        pltpu.VMEM((_BPW,), jnp.int32),
        pltpu.VMEM((_BPW, EMB), jnp.float32),
        pltpu.SemaphoreType.DMA,
    ],
)(_sc_gather_body)


_BT = 1024


def _mlp_body(eu_ref, ei_ref, nf_ref, w1u_ref, w1i_ref, w1n_ref, b1_ref,
              w2_ref, b2_ref, out_ref):
    h = jnp.dot(eu_ref[...], w1u_ref[...], preferred_element_type=jnp.float32)
    h += jnp.dot(ei_ref[...], w1i_ref[...], preferred_element_type=jnp.float32)
    h += jnp.dot(nf_ref[...], w1n_ref[...], preferred_element_type=jnp.float32)
    h += b1_ref[...]
    h = jnp.maximum(h, 0.0)
    o = jnp.dot(h, w2_ref[...], preferred_element_type=jnp.float32)
    o += b2_ref[...]
    o = 1.0 / (1.0 + jnp.exp(-o))
    out_ref[...] = o * (RATING_MAX - RATING_MIN) + RATING_MIN


def _mlp(eu, ei, nf, w1u, w1i, w1n, b1, w2, b2):
    grid = (BATCH // 2 // _BT,)
    args = (eu, ei, nf, w1u, w1i, w1n, b1, w2, b2)
    bspec_b = lambda shape: pl.BlockSpec((_BT,) + shape[1:],
                                         lambda i: (i,) + (0,) * (len(shape) - 1))
    full = lambda shape: pl.BlockSpec(shape, lambda i: (0,) * len(shape))
    in_specs = [bspec_b(a.shape) for a in args[:3]]
    in_specs += [full(a.shape) for a in args[3:]]
    return pl.pallas_call(
        _mlp_body,
        grid=grid,
        in_specs=in_specs,
        out_specs=pl.BlockSpec((_BT, 2), lambda i: (i, 0)),
        out_shape=jax.ShapeDtypeStruct((BATCH // 2, 2), jnp.float32),
    )(*args)


def _blockdiag(w):
    z = jnp.zeros_like(w)
    return jnp.concatenate([jnp.concatenate([w, z], 1),
                            jnp.concatenate([z, w], 1)], 0)


def kernel(X, user_table, item_table, W1, b1, W2, b2):
    uidx = X[:, 0]
    iidx = X[:, 1]
    nf = X[:, 2:].astype(jnp.float32)
    eu = _sc_gather(user_table, uidx)
    ei = _sc_gather(item_table, iidx)
    eup = eu.reshape(BATCH // 2, 2 * EMB)
    eip = ei.reshape(BATCH // 2, 2 * EMB)
    nfp = nf.reshape(BATCH // 2, 2 * N_NUM)
    w1u = _blockdiag(W1[:EMB])
    w1i = _blockdiag(W1[EMB:2 * EMB])
    w1n = _blockdiag(W1[2 * EMB:])
    w2bd = _blockdiag(W2)
    b1p = jnp.concatenate([b1, b1]).reshape(1, 2 * N_ACT)
    o = _mlp(eup, eip, nfp, w1u, w1i, w1n, b1p, w2bd,
             b2.reshape(1, 1))
    return o.reshape(BATCH, 1)
